# baseline (speedup 1.0000x reference)
import functools

import jax
import jax.numpy as jnp
from jax import lax
from jax.experimental import pallas as pl
from jax.experimental.pallas import tpu as pltpu
from jax.experimental.pallas import tpu_sc as plsc

NC = 2
NS = 16
L = 16
NW = NC * NS
EB = 128
ALPHA = 0.05
NEG_SLOPE = 0.2

_SC_PARAMS = dict(
    compiler_params=pltpu.CompilerParams(
        needs_layout_passes=False, use_tc_tiling_on_sc=False
    ),
)


def _pad_up(x, m):
    return ((x + m - 1) // m) * m


def _padrows(X, N):
    return jnp.pad(X, ((0, N - X.shape[0]), (0, 0)))


def _fc(X, W, b):
    def body(x_ref, w_ref, b_ref, o_ref):
        o_ref[...] = (
            jnp.dot(x_ref[...], w_ref[...], preferred_element_type=jnp.float32)
            + b_ref[...]
        )
    n, _ = X.shape
    H = W.shape[1]
    return pl.pallas_call(
        body, out_shape=jax.ShapeDtypeStruct((n, H), jnp.float32)
    )(X, W, b.reshape(1, H))


def _proj_attn1(X, W, al, ar):
    def body(x_ref, w_ref, al_ref, ar_ref, f_ref, el_ref, er_ref):
        f = jnp.dot(x_ref[...], w_ref[...], preferred_element_type=jnp.float32)
        f_ref[...] = f
        el_ref[...] = jnp.sum(f * al_ref[...], axis=1, keepdims=True)
        er_ref[...] = jnp.sum(f * ar_ref[...], axis=1, keepdims=True)
    n = X.shape[0]
    H = W.shape[1]
    return pl.pallas_call(
        body,
        out_shape=[
            jax.ShapeDtypeStruct((n, H), jnp.float32),
            jax.ShapeDtypeStruct((n, 1), jnp.float32),
            jax.ShapeDtypeStruct((n, 1), jnp.float32),
        ],
    )(X, W, al, ar)


def _proj_attn1_T(XT, W3, al, ar):
    heads, N, H = XT.shape
    C = W3.shape[2]
    def body(x_ref, w_ref, al_ref, ar_ref, f_ref, el_ref, er_ref):
        f = jnp.dot(x_ref[0], w_ref[0], preferred_element_type=jnp.float32)
        for h in range(1, heads):
            f += jnp.dot(x_ref[h], w_ref[h], preferred_element_type=jnp.float32)
        f_ref[...] = f
        el_ref[...] = jnp.sum(f * al_ref[...], axis=1, keepdims=True)
        er_ref[...] = jnp.sum(f * ar_ref[...], axis=1, keepdims=True)
    blk = N // 8
    return pl.pallas_call(
        body,
        grid=(N // blk,),
        in_specs=[
            pl.BlockSpec((heads, blk, H), lambda i: (0, i, 0)),
            pl.BlockSpec(W3.shape, lambda i: (0, 0, 0)),
            pl.BlockSpec(al.shape, lambda i: (0, 0)),
            pl.BlockSpec(ar.shape, lambda i: (0, 0)),
        ],
        out_specs=[
            pl.BlockSpec((blk, C), lambda i: (i, 0)),
            pl.BlockSpec((blk, 1), lambda i: (i, 0)),
            pl.BlockSpec((blk, 1), lambda i: (i, 0)),
        ],
        out_shape=[
            jax.ShapeDtypeStruct((N, C), jnp.float32),
            jax.ShapeDtypeStruct((N, 1), jnp.float32),
            jax.ShapeDtypeStruct((N, 1), jnp.float32),
        ],
    )(XT, W3, al, ar)


def _proj_heads(X, W3T):
    N, K = X.shape
    heads, H = W3T.shape[0], W3T.shape[2]
    blk = N // 8
    def body(x_ref, w_ref, o_ref):
        o_ref[0] = jnp.dot(
            x_ref[...], w_ref[0], preferred_element_type=jnp.float32
        )
    return pl.pallas_call(
        body,
        grid=(heads, N // blk),
        in_specs=[
            pl.BlockSpec((blk, K), lambda h, i: (i, 0)),
            pl.BlockSpec((1, K, H), lambda h, i: (h, 0, 0)),
        ],
        out_specs=pl.BlockSpec((1, blk, H), lambda h, i: (h, i, 0)),
        out_shape=jax.ShapeDtypeStruct((heads, N, H), jnp.float32),
    )(X, W3T)


def _proj_heads_T(XT, W4):
    heads, N, H = XT.shape
    blk = N // 8
    def body(x_ref, w_ref, o_ref):
        acc = jnp.dot(x_ref[0], w_ref[0, 0], preferred_element_type=jnp.float32)
        for hi in range(1, heads):
            acc += jnp.dot(
                x_ref[hi], w_ref[0, hi], preferred_element_type=jnp.float32
            )
        o_ref[0] = acc
    return pl.pallas_call(
        body,
        grid=(heads, N // blk),
        in_specs=[
            pl.BlockSpec((heads, blk, H), lambda h, i: (0, i, 0)),
            pl.BlockSpec((1, heads, H, H), lambda h, i: (h, 0, 0, 0)),
        ],
        out_specs=pl.BlockSpec((1, blk, H), lambda h, i: (h, i, 0)),
        out_shape=jax.ShapeDtypeStruct((heads, N, H), jnp.float32),
    )(XT, W4)


def _attn_logitsP(Xp, W3T, al, ar):
    N, K = Xp.shape
    heads = W3T.shape[0]
    blk = N // 8
    def body(x_ref, w_ref, al_ref, ar_ref, el_ref, er_ref):
        VlT = jnp.sum(w_ref[...] * al_ref[...][:, None, :], axis=-1)
        VrT = jnp.sum(w_ref[...] * ar_ref[...][:, None, :], axis=-1)
        el_ref[...] = jnp.dot(x_ref[...], VlT.T,
                              preferred_element_type=jnp.float32)
        er_ref[...] = jnp.dot(x_ref[...], VrT.T,
                              preferred_element_type=jnp.float32)
    return pl.pallas_call(
        body,
        grid=(N // blk,),
        in_specs=[
            pl.BlockSpec((blk, K), lambda i: (i, 0)),
            pl.BlockSpec((heads, K, W3T.shape[2]), lambda i: (0, 0, 0)),
            pl.BlockSpec(al.shape, lambda i: (0, 0)),
            pl.BlockSpec(ar.shape, lambda i: (0, 0)),
        ],
        out_specs=[
            pl.BlockSpec((blk, heads), lambda i: (i, 0)),
            pl.BlockSpec((blk, heads), lambda i: (i, 0)),
        ],
        out_shape=[
            jax.ShapeDtypeStruct((N, heads), jnp.float32),
            jax.ShapeDtypeStruct((N, heads), jnp.float32),
        ],
    )(Xp, W3T, al, ar)


def _attn_logits_TP(XT, W7, al, ar):
    NP, N, Dp = XT.shape
    heads = W7.shape[2]
    blk = N // 8
    def body(x_ref, w_ref, al_ref, ar_ref, el_ref, er_ref):
        Vl = jnp.sum(w_ref[...] * al_ref[...][None, None, :, :], axis=-1)
        Vr = jnp.sum(w_ref[...] * ar_ref[...][None, None, :, :], axis=-1)
        accl = jnp.dot(x_ref[0], Vl[0], preferred_element_type=jnp.float32)
        accr = jnp.dot(x_ref[0], Vr[0], preferred_element_type=jnp.float32)
        for pi in range(1, NP):
            accl += jnp.dot(x_ref[pi], Vl[pi],
                            preferred_element_type=jnp.float32)
            accr += jnp.dot(x_ref[pi], Vr[pi],
                            preferred_element_type=jnp.float32)
        el_ref[...] = accl
        er_ref[...] = accr
    return pl.pallas_call(
        body,
        grid=(N // blk,),
        in_specs=[
            pl.BlockSpec((NP, blk, Dp), lambda i: (0, i, 0)),
            pl.BlockSpec(W7.shape, lambda i: (0, 0, 0, 0)),
            pl.BlockSpec(al.shape, lambda i: (0, 0)),
            pl.BlockSpec(ar.shape, lambda i: (0, 0)),
        ],
        out_specs=[
            pl.BlockSpec((blk, heads), lambda i: (i, 0)),
            pl.BlockSpec((blk, heads), lambda i: (i, 0)),
        ],
        out_shape=[
            jax.ShapeDtypeStruct((N, heads), jnp.float32),
            jax.ShapeDtypeStruct((N, heads), jnp.float32),
        ],
    )(XT, W7, al, ar)


def _transpose8(x):
    N, heads = x.shape
    def body(x_ref, o_ref):
        o_ref[...] = x_ref[...].T
    return pl.pallas_call(
        body, out_shape=jax.ShapeDtypeStruct((heads, N), jnp.float32)
    )(x)


def _combine_inv(s_parts):
    def body(s_ref, o_ref):
        o_ref[...] = 1.0 / (jnp.sum(s_ref[...], axis=0, keepdims=True) + 1e-12)
    K = s_parts.shape[1]
    return pl.pallas_call(
        body, out_shape=jax.ShapeDtypeStruct((1, K), jnp.float32)
    )(s_parts)


def _resid1(p, X):
    def body(p_ref, x_ref, o_ref):
        o_ref[...] = p_ref[0] + p_ref[1] + x_ref[...]
    n, H = X.shape
    return pl.pallas_call(
        body, out_shape=jax.ShapeDtypeStruct((n, H), jnp.float32)
    )(p, X)


def _combine_relu(p, n, res=None):
    heads, N, H = p.shape[1], p.shape[2], p.shape[3]
    blk = N // 8
    def mask(i, val):
        rows = lax.broadcasted_iota(jnp.int32, (heads, blk, H), 1) + i * blk
        return jnp.where(rows < n, val, 0.0)
    if res is None:
        def body(p_ref, o_ref):
            i = pl.program_id(0)
            o_ref[...] = mask(i, jax.nn.relu(p_ref[0] + p_ref[1]))
        ins = (p,)
        in_specs = [pl.BlockSpec((2, heads, blk, H), lambda i: (0, 0, i, 0))]
    else:
        def body(p_ref, r_ref, o_ref):
            i = pl.program_id(0)
            o_ref[...] = mask(i, jax.nn.relu(p_ref[0] + p_ref[1] + r_ref[...]))
        ins = (p, res)
        in_specs = [
            pl.BlockSpec((2, heads, blk, H), lambda i: (0, 0, i, 0)),
            pl.BlockSpec((heads, blk, H), lambda i: (0, i, 0)),
        ]
    return pl.pallas_call(
        body,
        grid=(N // blk,),
        in_specs=in_specs,
        out_specs=pl.BlockSpec((heads, blk, H), lambda i: (0, i, 0)),
        out_shape=jax.ShapeDtypeStruct((heads, N, H), jnp.float32),
    )(*ins)


def _final(pf, XT, Wres3):
    heads, N, H = XT.shape
    C = Wres3.shape[2]
    def body(p_ref, x_ref, w_ref, o_ref):
        y = p_ref[0] + p_ref[1]
        for h in range(heads):
            y += jnp.dot(x_ref[h], w_ref[h], preferred_element_type=jnp.float32)
        nrm = jnp.sqrt(jnp.sum(y * y, axis=1, keepdims=True))
        o_ref[...] = y / jnp.maximum(nrm, 1e-12)
    blk = N // 8
    return pl.pallas_call(
        body,
        grid=(N // blk,),
        in_specs=[
            pl.BlockSpec((2, blk, C), lambda i: (0, i, 0)),
            pl.BlockSpec((heads, blk, H), lambda i: (0, i, 0)),
            pl.BlockSpec(Wres3.shape, lambda i: (0, 0, 0)),
        ],
        out_specs=pl.BlockSpec((blk, C), lambda i: (i, 0)),
        out_shape=jax.ShapeDtypeStruct((N, C), jnp.float32),
    )(pf, XT, Wres3)


def _mesh():
    return plsc.VectorSubcoreMesh(
        core_axis_name="c", subcore_axis_name="s", num_cores=NC, num_subcores=NS
    )


@functools.partial(jax.jit, static_argnames=("N_ACC", "heads"))
def _attn_stats(src, dst, elT, erT, *, N_ACC, heads):
    E_pad = src.shape[0]
    M = E_pad // NW
    nblk = M // EB

    @functools.partial(
        pl.kernel,
        out_type=[
            jax.ShapeDtypeStruct((heads * E_pad,), jnp.float32),
            jax.ShapeDtypeStruct((NW, heads * N_ACC), jnp.float32),
        ],
        mesh=_mesh(),
        scratch_types=[
            pltpu.VMEM((N_ACC,), jnp.float32),
            pltpu.VMEM((N_ACC,), jnp.float32),
            pltpu.VMEM((heads * N_ACC,), jnp.float32),
            pltpu.VMEM((EB,), jnp.int32),
            pltpu.VMEM((EB,), jnp.int32),
            pltpu.VMEM((EB,), jnp.float32),
        ],
        **_SC_PARAMS,
    )
    def k(src_h, dst_h, el_h, er_h, exT_h, sp_h, elv, erv, sv, srcb, dstb, exb):
        c = lax.axis_index("c")
        s = lax.axis_index("s")
        wid = s * NC + c
        base0 = wid * M

        @pl.loop(0, heads * N_ACC // L)
        def _zero(i):
            sv[pl.ds(i * L, L)] = jnp.zeros((L,), jnp.float32)

        for h in range(heads):
            pltpu.sync_copy(el_h.at[pl.ds(h * N_ACC, N_ACC)], elv)
            pltpu.sync_copy(er_h.at[pl.ds(h * N_ACC, N_ACC)], erv)

            @pl.loop(0, nblk)
            def _blk(b):
                base = base0 + b * EB
                pltpu.sync_copy(src_h.at[pl.ds(base, EB)], srcb)
                pltpu.sync_copy(dst_h.at[pl.ds(base, EB)], dstb)
                for j in range(EB // L):
                    sc_ = srcb[pl.ds(j * L, L)]
                    dc_ = dstb[pl.ds(j * L, L)]
                    e = (plsc.load_gather(elv, [sc_])
                         + plsc.load_gather(erv, [dc_]))
                    e = jnp.maximum(e, NEG_SLOPE * e)
                    ex = jnp.exp(e)
                    exb[pl.ds(j * L, L)] = ex
                    plsc.addupdate_scatter(sv, [dc_ + (h * N_ACC)], ex)
                pltpu.sync_copy(exb, exT_h.at[pl.ds(h * E_pad + base, EB)])

        pltpu.sync_copy(sv, sp_h.at[wid])

    return k(src, dst, elT, erT)


@functools.partial(jax.jit, static_argnames=("N_ACC", "heads", "blend"))
def _attn_norm(dst, exT, invs, a0T=None, *, N_ACC, heads, blend=False):
    E_pad = dst.shape[0]
    M = E_pad // NW
    nblk = M // EB

    @functools.partial(
        pl.kernel,
        out_type=jax.ShapeDtypeStruct((heads * E_pad,), jnp.float32),
        mesh=_mesh(),
        scratch_types=[
            pltpu.VMEM((N_ACC,), jnp.float32),
            pltpu.VMEM((EB,), jnp.int32),
            pltpu.VMEM((EB,), jnp.float32),
            pltpu.VMEM((EB,), jnp.float32),
            pltpu.VMEM((EB,), jnp.float32),
        ],
        **_SC_PARAMS,
    )
    def k(*refs):
        if blend:
            (dst_h, exT_h, invs_h, a0T_h) = refs[:4]
            (aT_h, invs_v, dstb, exb, a0b, ab) = refs[4:]
        else:
            (dst_h, exT_h, invs_h) = refs[:3]
            (aT_h, invs_v, dstb, exb, a0b, ab) = refs[3:]
            a0T_h = None
        c = lax.axis_index("c")
        s = lax.axis_index("s")
        wid = s * NC + c
        base0 = wid * M

        for h in range(heads):
            pltpu.sync_copy(invs_h.at[pl.ds(h * N_ACC, N_ACC)], invs_v)

            @pl.loop(0, nblk)
            def _blk(b):
                base = base0 + b * EB
                pltpu.sync_copy(dst_h.at[pl.ds(base, EB)], dstb)
                pltpu.sync_copy(exT_h.at[pl.ds(h * E_pad + base, EB)], exb)
                if blend:
                    pltpu.sync_copy(
                        a0T_h.at[pl.ds(h * E_pad + base, EB)], a0b)
                for j in range(EB // L):
                    dc_ = dstb[pl.ds(j * L, L)]
                    a = exb[pl.ds(j * L, L)] * plsc.load_gather(invs_v, [dc_])
                    if blend:
                        a = (a * (1.0 - ALPHA)
                             + ALPHA * a0b[pl.ds(j * L, L)])
                    ab[pl.ds(j * L, L)] = a
                pltpu.sync_copy(ab, aT_h.at[pl.ds(h * E_pad + base, EB)])

    args = (dst, exT, invs) + ((a0T,) if blend else ())
    return k(*args)


@functools.partial(jax.jit, static_argnames=("N_ACC", "heads", "D"))
def _msg_pass(src, dst, aT, featflat, *, N_ACC, heads, D):
    E_pad = src.shape[0]
    M = E_pad // NW
    nblk = M // EB
    rpt = N_ACC // NS
    HP = 2 if heads == 8 else 1
    NP = heads // HP
    Dp = HP * D

    scratch = [
        pltpu.VMEM_SHARED((N_ACC, Dp), jnp.float32),
        pltpu.VMEM((2, EB), jnp.int32),
        pltpu.VMEM((2, EB), jnp.int32),
        pltpu.VMEM((2, EB), jnp.int32),
        pltpu.VMEM((2, EB), jnp.int32),
        pltpu.VMEM((2, HP, EB), jnp.float32),
        pltpu.VMEM((2, EB, Dp), jnp.float32),
        pltpu.VMEM((32, Dp), jnp.float32),
        pltpu.SemaphoreType.DMA,
        pltpu.SemaphoreType.DMA,
        pltpu.SemaphoreType.DMA,
        pltpu.SemaphoreType.DMA,
        pltpu.SemaphoreType.DMA,
        pltpu.SemaphoreType.DMA,
    ]

    @functools.partial(
        pl.kernel,
        out_type=jax.ShapeDtypeStruct((NC, NP, N_ACC, Dp), jnp.float32),
        mesh=_mesh(), scratch_types=scratch,
        **_SC_PARAMS,
    )
    def k(src_h, dst_h, aT_h, feat_h, out_h, acc, srcb, dstb, idxb, sdst,
          ab, rows, zv, ls0, ls1, gs0, gs1, ss0, ss1):
        lsem = (ls0, ls1)
        gsem = (gs0, gs1)
        ssem = (ss0, ss1)

        c = lax.axis_index("c")
        s = lax.axis_index("s")
        wid = s * NC + c
        base0 = wid * M

        @pl.loop(0, 32)
        def _zfill(i):
            for j in range(Dp // L):
                zv[i, pl.ds(j * L, L)] = jnp.zeros((L,), jnp.float32)

        for p in range(NP):
            off = 0
            while off < rpt:
                csz = min(32, rpt - off)
                pltpu.sync_copy(
                    zv.at[pl.ds(0, csz), :],
                    acc.at[pl.ds(s * rpt + off, csz), :],
                )
                off += csz
            plsc.subcore_barrier()

            def loads(t, k_):
                base = base0 + k_ * EB
                pltpu.async_copy(src_h.at[pl.ds(base, EB)],
                                 srcb.at[t], lsem[t])
                pltpu.async_copy(dst_h.at[pl.ds(base, EB)],
                                 dstb.at[t], lsem[t])
                for hh in range(HP):
                    eo = (p * HP + hh) * E_pad + base
                    pltpu.async_copy(aT_h.at[pl.ds(eo, EB)],
                                     ab.at[t, hh], lsem[t])

            def wait_loads(t):
                pltpu.make_async_copy(
                    src_h.at[pl.ds(0, EB)], srcb.at[t], lsem[t]).wait()
                pltpu.make_async_copy(
                    dst_h.at[pl.ds(0, EB)], dstb.at[t], lsem[t]).wait()
                for hh in range(HP):
                    pltpu.make_async_copy(
                        aT_h.at[pl.ds(0, EB)], ab.at[t, hh], lsem[t]).wait()

            loads(0, 0)

            @pl.loop(0, nblk, step=2)
            def _blk(b):
                for t in range(2):
                    k_ = b + t
                    wait_loads(t)
                    @pl.when(k_ >= 2)
                    def _drain():
                        pltpu.make_async_copy(
                            rows.at[t], acc.at[sdst.at[t]], ssem[t]).wait()
                    if NP > 1:
                        for j in range(EB // L):
                            idxb[t, pl.ds(j * L, L)] = (
                                srcb[t, pl.ds(j * L, L)] + (p * N_ACC)
                            )
                        gref = idxb.at[t]
                    else:
                        gref = srcb.at[t]
                    gd = pltpu.async_copy(feat_h.at[gref], rows.at[t],
                                          gsem[t])

                    @pl.when(k_ + 1 < nblk)
                    def _pref():
                        loads(1 - t, k_ + 1)

                    for j in range(EB // L):
                        sdst[t, pl.ds(j * L, L)] = dstb[t, pl.ds(j * L, L)]
                    gd.wait()

                    rv = rows.at[t]

                    @pl.loop(0, EB, unroll=8)
                    def _scale(bb):
                        for hh in range(HP):
                            av = plsc.load_gather(
                                ab.at[t, hh],
                                [jnp.full((L,), bb, jnp.int32)])
                            for j in range(D // L):
                                o = hh * D + j * L
                                rv[bb, pl.ds(o, L)] = rv[bb, pl.ds(o, L)] * av

                    pltpu.async_copy(rows.at[t], acc.at[sdst.at[t]],
                                     ssem[t], add=True)

            for t in range(2):
                pltpu.make_async_copy(
                    rows.at[t], acc.at[sdst.at[t]], ssem[t]).wait()

            plsc.subcore_barrier()
            pltpu.sync_copy(
                acc.at[pl.ds(s * rpt, rpt), :],
                out_h.at[c, p, pl.ds(s * rpt, rpt), :],
            )
            plsc.subcore_barrier()

    return k(src, dst, aT, featflat)


def _pad_edges(src, dst, n):
    E = src.shape[0]
    E_pad = _pad_up(E, NW * EB * 2)
    pad = E_pad - E
    src_p = jnp.concatenate([src, jnp.zeros((pad,), jnp.int32)])
    dst_p = jnp.concatenate([dst, jnp.full((pad,), n, jnp.int32)])
    return src_p, dst_p


def _gat_layer1(featp, elf, erf, src, dst, n, N_ACC):
    src_p, dst_p = _pad_edges(src, dst, n)
    exT, s_parts = _attn_stats(src_p, dst_p, elf, erf, N_ACC=N_ACC, heads=1)
    invs = _combine_inv(s_parts).reshape(-1)
    aT = _attn_norm(dst_p, exT, invs, N_ACC=N_ACC, heads=1)
    return _msg_pass(src_p, dst_p, aT, featp,
                     N_ACC=N_ACC, heads=1, D=featp.shape[1])


def kernel(features_0, features_1, features_2, W_fc0, b_fc0, W_fc1, b_fc1,
           W_fc2, b_fc2, Wp0, alp0, arp0, Wp1, alp1, arp1, Wg0, al0, ar0,
           Wg1, al1, ar1, Wgf, alf, arf, Wres, hg0_src, hg0_dst, hg1_src,
           hg1_dst, g_src, g_dst):
    H = 64
    heads = 8
    n = 10000
    NA0 = _pad_up(5000 + 1, 128)
    NA1 = _pad_up(8000 + 1, 128)
    NAg = _pad_up(n + 1, 128)

    h0 = _fc(features_0, W_fc0, b_fc0)
    h1_ = _fc(features_1, W_fc1, b_fc1)
    h2_ = _fc(features_2, W_fc2, b_fc2)

    feat, el, er = _proj_attn1(_padrows(h0, NA0), Wp0, alp0, arp0)
    p = _gat_layer1(feat, el.reshape(-1), er.reshape(-1),
                    hg0_src, hg0_dst, 5000, NA0)
    h0n = _resid1(p[:, 0, :5000], h0)

    hcat = jnp.concatenate([h0n, h1_], axis=0)
    feat, el, er = _proj_attn1(_padrows(hcat, NA1), Wp1, alp1, arp1)
    p = _gat_layer1(feat, el.reshape(-1), er.reshape(-1),
                    hg1_src, hg1_dst, 8000, NA1)
    h1n = _resid1(p[:, 0, 5000:8000], h1_)

    hfp = _padrows(jnp.concatenate([h0n, h1n, h2_], axis=0), NAg)
    src_p, dst_p = _pad_edges(g_src, g_dst, n)

    NP = heads // 2
    Dp = 2 * H
    W3T = jnp.transpose(Wg0.reshape(H, heads, H), (1, 0, 2))
    W5 = jnp.transpose(Wg0.reshape(H, NP, 2, H), (1, 0, 2, 3)).reshape(
        NP, H, Dp)
    featP = _proj_heads(hfp, W5)
    el, er = _attn_logitsP(hfp, W3T, al0, ar0)
    elT, erT = _transpose8(el), _transpose8(er)
    exT, s_parts = _attn_stats(src_p, dst_p, elT.reshape(-1), erT.reshape(-1),
                               N_ACC=NAg, heads=heads)
    invs = _combine_inv(s_parts).reshape(-1)
    a0T = _attn_norm(dst_p, exT, invs, N_ACC=NAg, heads=heads)
    parts = _msg_pass(src_p, dst_p, a0T, featP.reshape(NP * NAg, Dp),
                      N_ACC=NAg, heads=heads, D=H)
    h1g = _combine_relu(parts, n)

    W6 = jnp.transpose(
        Wg1.reshape(NP, 2, H, NP, 2, H), (3, 0, 1, 2, 4, 5)
    ).reshape(NP, NP, Dp, Dp)
    W7 = Wg1.reshape(NP, Dp, heads, H)
    featP = _proj_heads_T(h1g, W6)
    el, er = _attn_logits_TP(h1g, W7, al1, ar1)
    elT, erT = _transpose8(el), _transpose8(er)
    exT, s_parts = _attn_stats(src_p, dst_p, elT.reshape(-1), erT.reshape(-1),
                               N_ACC=NAg, heads=heads)
    invs = _combine_inv(s_parts).reshape(-1)
    aT1 = _attn_norm(dst_p, exT, invs, a0T, N_ACC=NAg, heads=heads,
                     blend=True)
    parts = _msg_pass(src_p, dst_p, aT1, featP.reshape(NP * NAg, Dp),
                      N_ACC=NAg, heads=heads, D=H)
    h2g = _combine_relu(parts, n, h1g)

    C = Wgf.shape[1]
    featf, el, er = _proj_attn1_T(h2g, Wgf.reshape(NP, Dp, C), alf, arf)
    exT, s_parts = _attn_stats(src_p, dst_p, el.reshape(-1), er.reshape(-1),
                               N_ACC=NAg, heads=1)
    invs = _combine_inv(s_parts).reshape(-1)
    aTf = _attn_norm(dst_p, exT, invs, N_ACC=NAg, heads=1)
    partsf = _msg_pass(src_p, dst_p, aTf, featf, N_ACC=NAg, heads=1, D=C)
    out = _final(partsf[:, 0], h2g, Wres.reshape(NP, Dp, C))
    return out[:n]


# speedup vs baseline: 1.2071x; 1.2071x over previous
import functools

import jax
import jax.numpy as jnp
from jax import lax
from jax.experimental import pallas as pl
from jax.experimental.pallas import tpu as pltpu
from jax.experimental.pallas import tpu_sc as plsc

NC = 2
NS = 16
L = 16
NW = NC * NS
EB = 128
ALPHA = 0.05
NEG_SLOPE = 0.2

_SC_PARAMS = dict(
    compiler_params=pltpu.CompilerParams(
        needs_layout_passes=False, use_tc_tiling_on_sc=False
    ),
)


def _pad_up(x, m):
    return ((x + m - 1) // m) * m


def _padrows(X, N):
    return jnp.pad(X, ((0, N - X.shape[0]), (0, 0)))


def _fc(X, W, b):
    def body(x_ref, w_ref, b_ref, o_ref):
        o_ref[...] = (
            jnp.dot(x_ref[...], w_ref[...], preferred_element_type=jnp.float32)
            + b_ref[...]
        )
    n, _ = X.shape
    H = W.shape[1]
    return pl.pallas_call(
        body, out_shape=jax.ShapeDtypeStruct((n, H), jnp.float32)
    )(X, W, b.reshape(1, H))


def _proj_attn1(X, W, al, ar):
    def body(x_ref, w_ref, al_ref, ar_ref, f_ref, el_ref, er_ref):
        f = jnp.dot(x_ref[...], w_ref[...], preferred_element_type=jnp.float32)
        f_ref[...] = f
        el_ref[...] = jnp.sum(f * al_ref[...], axis=1, keepdims=True)
        er_ref[...] = jnp.sum(f * ar_ref[...], axis=1, keepdims=True)
    n = X.shape[0]
    H = W.shape[1]
    return pl.pallas_call(
        body,
        out_shape=[
            jax.ShapeDtypeStruct((n, H), jnp.float32),
            jax.ShapeDtypeStruct((n, 1), jnp.float32),
            jax.ShapeDtypeStruct((n, 1), jnp.float32),
        ],
    )(X, W, al, ar)


def _proj_attn1_T(XT, W3, al, ar):
    heads, N, H = XT.shape
    C = W3.shape[2]
    def body(x_ref, w_ref, al_ref, ar_ref, f_ref, el_ref, er_ref):
        f = jnp.dot(x_ref[0], w_ref[0], preferred_element_type=jnp.float32)
        for h in range(1, heads):
            f += jnp.dot(x_ref[h], w_ref[h], preferred_element_type=jnp.float32)
        f_ref[...] = f
        el_ref[...] = jnp.sum(f * al_ref[...], axis=1, keepdims=True)
        er_ref[...] = jnp.sum(f * ar_ref[...], axis=1, keepdims=True)
    blk = N // 8
    return pl.pallas_call(
        body,
        grid=(N // blk,),
        in_specs=[
            pl.BlockSpec((heads, blk, H), lambda i: (0, i, 0)),
            pl.BlockSpec(W3.shape, lambda i: (0, 0, 0)),
            pl.BlockSpec(al.shape, lambda i: (0, 0)),
            pl.BlockSpec(ar.shape, lambda i: (0, 0)),
        ],
        out_specs=[
            pl.BlockSpec((blk, C), lambda i: (i, 0)),
            pl.BlockSpec((blk, 1), lambda i: (i, 0)),
            pl.BlockSpec((blk, 1), lambda i: (i, 0)),
        ],
        out_shape=[
            jax.ShapeDtypeStruct((N, C), jnp.float32),
            jax.ShapeDtypeStruct((N, 1), jnp.float32),
            jax.ShapeDtypeStruct((N, 1), jnp.float32),
        ],
    )(XT, W3, al, ar)


def _proj_heads(X, W3T):
    N, K = X.shape
    heads, H = W3T.shape[0], W3T.shape[2]
    blk = N // 8
    def body(x_ref, w_ref, o_ref):
        o_ref[0] = jnp.dot(
            x_ref[...], w_ref[0], preferred_element_type=jnp.float32
        )
    return pl.pallas_call(
        body,
        grid=(heads, N // blk),
        in_specs=[
            pl.BlockSpec((blk, K), lambda h, i: (i, 0)),
            pl.BlockSpec((1, K, H), lambda h, i: (h, 0, 0)),
        ],
        out_specs=pl.BlockSpec((1, blk, H), lambda h, i: (h, i, 0)),
        out_shape=jax.ShapeDtypeStruct((heads, N, H), jnp.float32),
    )(X, W3T)


def _proj_heads_T(XT, W4):
    heads, N, H = XT.shape
    blk = N // 8
    def body(x_ref, w_ref, o_ref):
        acc = jnp.dot(x_ref[0], w_ref[0, 0], preferred_element_type=jnp.float32)
        for hi in range(1, heads):
            acc += jnp.dot(
                x_ref[hi], w_ref[0, hi], preferred_element_type=jnp.float32
            )
        o_ref[0] = acc
    return pl.pallas_call(
        body,
        grid=(heads, N // blk),
        in_specs=[
            pl.BlockSpec((heads, blk, H), lambda h, i: (0, i, 0)),
            pl.BlockSpec((1, heads, H, H), lambda h, i: (h, 0, 0, 0)),
        ],
        out_specs=pl.BlockSpec((1, blk, H), lambda h, i: (h, i, 0)),
        out_shape=jax.ShapeDtypeStruct((heads, N, H), jnp.float32),
    )(XT, W4)


def _attn_logitsP(Xp, W3T, al, ar):
    N, K = Xp.shape
    heads = W3T.shape[0]
    blk = N // 8
    def body(x_ref, w_ref, al_ref, ar_ref, el_ref, er_ref):
        VlT = jnp.sum(w_ref[...] * al_ref[...][:, None, :], axis=-1)
        VrT = jnp.sum(w_ref[...] * ar_ref[...][:, None, :], axis=-1)
        el_ref[...] = jnp.dot(x_ref[...], VlT.T,
                              preferred_element_type=jnp.float32)
        er_ref[...] = jnp.dot(x_ref[...], VrT.T,
                              preferred_element_type=jnp.float32)
    return pl.pallas_call(
        body,
        grid=(N // blk,),
        in_specs=[
            pl.BlockSpec((blk, K), lambda i: (i, 0)),
            pl.BlockSpec((heads, K, W3T.shape[2]), lambda i: (0, 0, 0)),
            pl.BlockSpec(al.shape, lambda i: (0, 0)),
            pl.BlockSpec(ar.shape, lambda i: (0, 0)),
        ],
        out_specs=[
            pl.BlockSpec((blk, heads), lambda i: (i, 0)),
            pl.BlockSpec((blk, heads), lambda i: (i, 0)),
        ],
        out_shape=[
            jax.ShapeDtypeStruct((N, heads), jnp.float32),
            jax.ShapeDtypeStruct((N, heads), jnp.float32),
        ],
    )(Xp, W3T, al, ar)


def _attn_logits_TP(XT, W7, al, ar):
    NP, N, Dp = XT.shape
    heads = W7.shape[2]
    blk = N // 8
    def body(x_ref, w_ref, al_ref, ar_ref, el_ref, er_ref):
        Vl = jnp.sum(w_ref[...] * al_ref[...][None, None, :, :], axis=-1)
        Vr = jnp.sum(w_ref[...] * ar_ref[...][None, None, :, :], axis=-1)
        accl = jnp.dot(x_ref[0], Vl[0], preferred_element_type=jnp.float32)
        accr = jnp.dot(x_ref[0], Vr[0], preferred_element_type=jnp.float32)
        for pi in range(1, NP):
            accl += jnp.dot(x_ref[pi], Vl[pi],
                            preferred_element_type=jnp.float32)
            accr += jnp.dot(x_ref[pi], Vr[pi],
                            preferred_element_type=jnp.float32)
        el_ref[...] = accl
        er_ref[...] = accr
    return pl.pallas_call(
        body,
        grid=(N // blk,),
        in_specs=[
            pl.BlockSpec((NP, blk, Dp), lambda i: (0, i, 0)),
            pl.BlockSpec(W7.shape, lambda i: (0, 0, 0, 0)),
            pl.BlockSpec(al.shape, lambda i: (0, 0)),
            pl.BlockSpec(ar.shape, lambda i: (0, 0)),
        ],
        out_specs=[
            pl.BlockSpec((blk, heads), lambda i: (i, 0)),
            pl.BlockSpec((blk, heads), lambda i: (i, 0)),
        ],
        out_shape=[
            jax.ShapeDtypeStruct((N, heads), jnp.float32),
            jax.ShapeDtypeStruct((N, heads), jnp.float32),
        ],
    )(XT, W7, al, ar)


def _transpose8(x):
    N, heads = x.shape
    def body(x_ref, o_ref):
        o_ref[...] = x_ref[...].T
    return pl.pallas_call(
        body, out_shape=jax.ShapeDtypeStruct((heads, N), jnp.float32)
    )(x)


def _combine_inv(s_parts):
    def body(s_ref, o_ref):
        o_ref[...] = 1.0 / (jnp.sum(s_ref[...], axis=0, keepdims=True) + 1e-12)
    K = s_parts.shape[1]
    return pl.pallas_call(
        body, out_shape=jax.ShapeDtypeStruct((1, K), jnp.float32)
    )(s_parts)


def _resid1(p, X):
    def body(p_ref, x_ref, o_ref):
        o_ref[...] = p_ref[0] + p_ref[1] + x_ref[...]
    n, H = X.shape
    return pl.pallas_call(
        body, out_shape=jax.ShapeDtypeStruct((n, H), jnp.float32)
    )(p, X)


def _combine_relu(p, n, res=None):
    heads, N, H = p.shape[1], p.shape[2], p.shape[3]
    blk = N // 8
    def mask(i, val):
        rows = lax.broadcasted_iota(jnp.int32, (heads, blk, H), 1) + i * blk
        return jnp.where(rows < n, val, 0.0)
    if res is None:
        def body(p_ref, o_ref):
            i = pl.program_id(0)
            o_ref[...] = mask(i, jax.nn.relu(p_ref[0] + p_ref[1]))
        ins = (p,)
        in_specs = [pl.BlockSpec((2, heads, blk, H), lambda i: (0, 0, i, 0))]
    else:
        def body(p_ref, r_ref, o_ref):
            i = pl.program_id(0)
            o_ref[...] = mask(i, jax.nn.relu(p_ref[0] + p_ref[1] + r_ref[...]))
        ins = (p, res)
        in_specs = [
            pl.BlockSpec((2, heads, blk, H), lambda i: (0, 0, i, 0)),
            pl.BlockSpec((heads, blk, H), lambda i: (0, i, 0)),
        ]
    return pl.pallas_call(
        body,
        grid=(N // blk,),
        in_specs=in_specs,
        out_specs=pl.BlockSpec((heads, blk, H), lambda i: (0, i, 0)),
        out_shape=jax.ShapeDtypeStruct((heads, N, H), jnp.float32),
    )(*ins)


def _final(pf, XT, Wres3):
    heads, N, H = XT.shape
    C = Wres3.shape[2]
    def body(p_ref, x_ref, w_ref, o_ref):
        y = p_ref[0] + p_ref[1]
        for h in range(heads):
            y += jnp.dot(x_ref[h], w_ref[h], preferred_element_type=jnp.float32)
        nrm = jnp.sqrt(jnp.sum(y * y, axis=1, keepdims=True))
        o_ref[...] = y / jnp.maximum(nrm, 1e-12)
    blk = N // 8
    return pl.pallas_call(
        body,
        grid=(N // blk,),
        in_specs=[
            pl.BlockSpec((2, blk, C), lambda i: (0, i, 0)),
            pl.BlockSpec((heads, blk, H), lambda i: (0, i, 0)),
            pl.BlockSpec(Wres3.shape, lambda i: (0, 0, 0)),
        ],
        out_specs=pl.BlockSpec((blk, C), lambda i: (i, 0)),
        out_shape=jax.ShapeDtypeStruct((N, C), jnp.float32),
    )(pf, XT, Wres3)


def _mesh():
    return plsc.VectorSubcoreMesh(
        core_axis_name="c", subcore_axis_name="s", num_cores=NC, num_subcores=NS
    )


@functools.partial(jax.jit, static_argnames=("N_ACC", "heads"))
def _attn_stats(src, dst, elT, erT, *, N_ACC, heads):
    E_pad = src.shape[0]
    M = E_pad // NW
    nblk = M // EB

    @functools.partial(
        pl.kernel,
        out_type=[
            jax.ShapeDtypeStruct((heads * E_pad,), jnp.float32),
            jax.ShapeDtypeStruct((NW, heads * N_ACC), jnp.float32),
        ],
        mesh=_mesh(),
        scratch_types=[
            pltpu.VMEM((N_ACC,), jnp.float32),
            pltpu.VMEM((N_ACC,), jnp.float32),
            pltpu.VMEM((heads * N_ACC,), jnp.float32),
            pltpu.VMEM((EB,), jnp.int32),
            pltpu.VMEM((EB,), jnp.int32),
            pltpu.VMEM((EB,), jnp.float32),
        ],
        **_SC_PARAMS,
    )
    def k(src_h, dst_h, el_h, er_h, exT_h, sp_h, elv, erv, sv, srcb, dstb, exb):
        c = lax.axis_index("c")
        s = lax.axis_index("s")
        wid = s * NC + c
        base0 = wid * M

        @pl.loop(0, heads * N_ACC // L)
        def _zero(i):
            sv[pl.ds(i * L, L)] = jnp.zeros((L,), jnp.float32)

        for h in range(heads):
            pltpu.sync_copy(el_h.at[pl.ds(h * N_ACC, N_ACC)], elv)
            pltpu.sync_copy(er_h.at[pl.ds(h * N_ACC, N_ACC)], erv)

            @pl.loop(0, nblk)
            def _blk(b):
                base = base0 + b * EB
                pltpu.sync_copy(src_h.at[pl.ds(base, EB)], srcb)
                pltpu.sync_copy(dst_h.at[pl.ds(base, EB)], dstb)
                for j in range(EB // L):
                    sc_ = srcb[pl.ds(j * L, L)]
                    dc_ = dstb[pl.ds(j * L, L)]
                    e = (plsc.load_gather(elv, [sc_])
                         + plsc.load_gather(erv, [dc_]))
                    e = jnp.maximum(e, NEG_SLOPE * e)
                    ex = jnp.exp(e)
                    exb[pl.ds(j * L, L)] = ex
                    plsc.addupdate_scatter(sv, [dc_ + (h * N_ACC)], ex)
                g_ = wid * nblk + b
                pltpu.sync_copy(
                    exb, exT_h.at[pl.ds((g_ * heads + h) * EB, EB)])

        pltpu.sync_copy(sv, sp_h.at[wid])

    return k(src, dst, elT, erT)


@functools.partial(jax.jit, static_argnames=("N_ACC", "heads", "blend"))
def _attn_norm(dst, exT, invs, a0T=None, *, N_ACC, heads, blend=False):
    E_pad = dst.shape[0]
    M = E_pad // NW
    nblk = M // EB

    HB = heads * EB

    @functools.partial(
        pl.kernel,
        out_type=jax.ShapeDtypeStruct((heads * E_pad,), jnp.float32),
        mesh=_mesh(),
        scratch_types=[
            pltpu.VMEM((heads, N_ACC), jnp.float32),
            pltpu.VMEM((2, EB), jnp.int32),
            pltpu.VMEM((2, HB), jnp.float32),
            pltpu.VMEM((2, HB), jnp.float32),
            pltpu.VMEM((2, HB), jnp.float32),
            pltpu.SemaphoreType.DMA,
            pltpu.SemaphoreType.DMA,
            pltpu.SemaphoreType.DMA,
            pltpu.SemaphoreType.DMA,
        ],
        **_SC_PARAMS,
    )
    def k(*refs):
        if blend:
            (dst_h, exT_h, invs_h, a0T_h) = refs[:4]
            (aT_h, invs_v, dstb, exb, a0b, ab, ls0, ls1, ws0, ws1) = refs[4:]
        else:
            (dst_h, exT_h, invs_h) = refs[:3]
            (aT_h, invs_v, dstb, exb, a0b, ab, ls0, ls1, ws0, ws1) = refs[3:]
            a0T_h = None
        lsem = (ls0, ls1)
        wsem = (ws0, ws1)
        c = lax.axis_index("c")
        s = lax.axis_index("s")
        wid = s * NC + c
        g0_ = wid * nblk

        for h in range(heads):
            pltpu.sync_copy(invs_h.at[pl.ds(h * N_ACC, N_ACC)],
                            invs_v.at[h])

        def loads(t, k_):
            pltpu.async_copy(dst_h.at[pl.ds((g0_ + k_) * EB, EB)],
                             dstb.at[t], lsem[t])
            pltpu.async_copy(exT_h.at[pl.ds((g0_ + k_) * HB, HB)],
                             exb.at[t], lsem[t])
            if blend:
                pltpu.async_copy(a0T_h.at[pl.ds((g0_ + k_) * HB, HB)],
                                 a0b.at[t], lsem[t])

        def wait_loads(t):
            pltpu.make_async_copy(
                dst_h.at[pl.ds(0, EB)], dstb.at[t], lsem[t]).wait()
            pltpu.make_async_copy(
                exT_h.at[pl.ds(0, HB)], exb.at[t], lsem[t]).wait()
            if blend:
                pltpu.make_async_copy(
                    a0T_h.at[pl.ds(0, HB)], a0b.at[t], lsem[t]).wait()

        loads(0, 0)

        @pl.loop(0, nblk, step=2)
        def _blk(b):
            for t in range(2):
                k_ = b + t
                wait_loads(t)
                @pl.when(k_ >= 2)
                def _drain():
                    pltpu.make_async_copy(
                        ab.at[t], aT_h.at[pl.ds(0, HB)], wsem[t]).wait()

                @pl.when(k_ + 1 < nblk)
                def _pref():
                    loads(1 - t, k_ + 1)

                for j in range(EB // L):
                    dc_ = dstb[t, pl.ds(j * L, L)]
                    for h in range(heads):
                        iv = plsc.load_gather(
                            invs_v, [jnp.full((L,), h, jnp.int32), dc_])
                        a = exb[t, pl.ds(h * EB + j * L, L)] * iv
                        if blend:
                            a = (a * (1.0 - ALPHA)
                                 + ALPHA * a0b[t, pl.ds(h * EB + j * L, L)])
                        ab[t, pl.ds(h * EB + j * L, L)] = a
                pltpu.async_copy(ab.at[t],
                                 aT_h.at[pl.ds((g0_ + k_) * HB, HB)],
                                 wsem[t])

        for t in range(2):
            pltpu.make_async_copy(
                ab.at[t], aT_h.at[pl.ds(0, HB)], wsem[t]).wait()

    args = (dst, exT, invs) + ((a0T,) if blend else ())
    return k(*args)


@functools.partial(jax.jit, static_argnames=("N_ACC", "heads", "D"))
def _msg_pass(src, dst, aT, featflat, *, N_ACC, heads, D):
    E_pad = src.shape[0]
    M = E_pad // NW
    nblk = M // EB
    rpt = N_ACC // NS
    HP = 2 if heads == 8 else 1
    NP = heads // HP
    Dp = HP * D

    scratch = [
        pltpu.VMEM_SHARED((N_ACC, Dp), jnp.float32),
        pltpu.VMEM((2, EB), jnp.int32),
        pltpu.VMEM((2, EB), jnp.int32),
        pltpu.VMEM((2, EB), jnp.int32),
        pltpu.VMEM((2, EB), jnp.int32),
        pltpu.VMEM((2, HP * EB), jnp.float32),
        pltpu.VMEM((2, EB, Dp), jnp.float32),
        pltpu.VMEM((32, Dp), jnp.float32),
        pltpu.SemaphoreType.DMA,
        pltpu.SemaphoreType.DMA,
        pltpu.SemaphoreType.DMA,
        pltpu.SemaphoreType.DMA,
        pltpu.SemaphoreType.DMA,
        pltpu.SemaphoreType.DMA,
    ]

    @functools.partial(
        pl.kernel,
        out_type=jax.ShapeDtypeStruct((NC, NP, N_ACC, Dp), jnp.float32),
        mesh=_mesh(), scratch_types=scratch,
        **_SC_PARAMS,
    )
    def k(src_h, dst_h, aT_h, feat_h, out_h, acc, srcb, dstb, idxb, sdst,
          ab, rows, zv, ls0, ls1, gs0, gs1, ss0, ss1):
        lsem = (ls0, ls1)
        gsem = (gs0, gs1)
        ssem = (ss0, ss1)

        c = lax.axis_index("c")
        s = lax.axis_index("s")
        wid = s * NC + c
        base0 = wid * M

        @pl.loop(0, 32)
        def _zfill(i):
            for j in range(Dp // L):
                zv[i, pl.ds(j * L, L)] = jnp.zeros((L,), jnp.float32)

        for p in range(NP):
            off = 0
            while off < rpt:
                csz = min(32, rpt - off)
                pltpu.sync_copy(
                    zv.at[pl.ds(0, csz), :],
                    acc.at[pl.ds(s * rpt + off, csz), :],
                )
                off += csz
            plsc.subcore_barrier()

            def loads(t, k_):
                base = base0 + k_ * EB
                pltpu.async_copy(src_h.at[pl.ds(base, EB)],
                                 srcb.at[t], lsem[t])
                pltpu.async_copy(dst_h.at[pl.ds(base, EB)],
                                 dstb.at[t], lsem[t])
                eo = ((wid * nblk + k_) * heads + p * HP) * EB
                pltpu.async_copy(aT_h.at[pl.ds(eo, HP * EB)],
                                 ab.at[t], lsem[t])

            def wait_loads(t):
                pltpu.make_async_copy(
                    src_h.at[pl.ds(0, EB)], srcb.at[t], lsem[t]).wait()
                pltpu.make_async_copy(
                    dst_h.at[pl.ds(0, EB)], dstb.at[t], lsem[t]).wait()
                pltpu.make_async_copy(
                    aT_h.at[pl.ds(0, HP * EB)], ab.at[t], lsem[t]).wait()

            loads(0, 0)

            @pl.loop(0, nblk, step=2)
            def _blk(b):
                for t in range(2):
                    k_ = b + t
                    wait_loads(t)
                    @pl.when(k_ >= 2)
                    def _drain():
                        pltpu.make_async_copy(
                            rows.at[t], acc.at[sdst.at[t]], ssem[t]).wait()
                    if NP > 1:
                        for j in range(EB // L):
                            idxb[t, pl.ds(j * L, L)] = (
                                srcb[t, pl.ds(j * L, L)] + (p * N_ACC)
                            )
                        gref = idxb.at[t]
                    else:
                        gref = srcb.at[t]
                    gd = pltpu.async_copy(feat_h.at[gref], rows.at[t],
                                          gsem[t])

                    @pl.when(k_ + 1 < nblk)
                    def _pref():
                        loads(1 - t, k_ + 1)

                    for j in range(EB // L):
                        sdst[t, pl.ds(j * L, L)] = dstb[t, pl.ds(j * L, L)]
                    gd.wait()

                    rv = rows.at[t]

                    @pl.loop(0, EB, unroll=8)
                    def _scale(bb):
                        for hh in range(HP):
                            av = plsc.load_gather(
                                ab.at[t],
                                [jnp.full((L,), bb, jnp.int32) + (hh * EB)])
                            for j in range(D // L):
                                o = hh * D + j * L
                                rv[bb, pl.ds(o, L)] = rv[bb, pl.ds(o, L)] * av

                    pltpu.async_copy(rows.at[t], acc.at[sdst.at[t]],
                                     ssem[t], add=True)

            for t in range(2):
                pltpu.make_async_copy(
                    rows.at[t], acc.at[sdst.at[t]], ssem[t]).wait()

            plsc.subcore_barrier()
            pltpu.sync_copy(
                acc.at[pl.ds(s * rpt, rpt), :],
                out_h.at[c, p, pl.ds(s * rpt, rpt), :],
            )
            plsc.subcore_barrier()

    return k(src, dst, aT, featflat)


def _pad_edges(src, dst, n):
    E = src.shape[0]
    E_pad = _pad_up(E, NW * EB * 2)
    pad = E_pad - E
    src_p = jnp.concatenate([src, jnp.zeros((pad,), jnp.int32)])
    dst_p = jnp.concatenate([dst, jnp.full((pad,), n, jnp.int32)])
    return src_p, dst_p


def _gat_layer1(featp, elf, erf, src, dst, n, N_ACC):
    src_p, dst_p = _pad_edges(src, dst, n)
    exT, s_parts = _attn_stats(src_p, dst_p, elf, erf, N_ACC=N_ACC, heads=1)
    invs = _combine_inv(s_parts).reshape(-1)
    aT = _attn_norm(dst_p, exT, invs, N_ACC=N_ACC, heads=1)
    return _msg_pass(src_p, dst_p, aT, featp,
                     N_ACC=N_ACC, heads=1, D=featp.shape[1])


def kernel(features_0, features_1, features_2, W_fc0, b_fc0, W_fc1, b_fc1,
           W_fc2, b_fc2, Wp0, alp0, arp0, Wp1, alp1, arp1, Wg0, al0, ar0,
           Wg1, al1, ar1, Wgf, alf, arf, Wres, hg0_src, hg0_dst, hg1_src,
           hg1_dst, g_src, g_dst):
    H = 64
    heads = 8
    n = 10000
    NA0 = _pad_up(5000 + 1, 128)
    NA1 = _pad_up(8000 + 1, 128)
    NAg = _pad_up(n + 1, 128)

    h0 = _fc(features_0, W_fc0, b_fc0)
    h1_ = _fc(features_1, W_fc1, b_fc1)
    h2_ = _fc(features_2, W_fc2, b_fc2)

    feat, el, er = _proj_attn1(_padrows(h0, NA0), Wp0, alp0, arp0)
    p = _gat_layer1(feat, el.reshape(-1), er.reshape(-1),
                    hg0_src, hg0_dst, 5000, NA0)
    h0n = _resid1(p[:, 0, :5000], h0)

    hcat = jnp.concatenate([h0n, h1_], axis=0)
    feat, el, er = _proj_attn1(_padrows(hcat, NA1), Wp1, alp1, arp1)
    p = _gat_layer1(feat, el.reshape(-1), er.reshape(-1),
                    hg1_src, hg1_dst, 8000, NA1)
    h1n = _resid1(p[:, 0, 5000:8000], h1_)

    hfp = _padrows(jnp.concatenate([h0n, h1n, h2_], axis=0), NAg)
    src_p, dst_p = _pad_edges(g_src, g_dst, n)

    NP = heads // 2
    Dp = 2 * H
    W3T = jnp.transpose(Wg0.reshape(H, heads, H), (1, 0, 2))
    W5 = jnp.transpose(Wg0.reshape(H, NP, 2, H), (1, 0, 2, 3)).reshape(
        NP, H, Dp)
    featP = _proj_heads(hfp, W5)
    el, er = _attn_logitsP(hfp, W3T, al0, ar0)
    elT, erT = _transpose8(el), _transpose8(er)
    exT, s_parts = _attn_stats(src_p, dst_p, elT.reshape(-1), erT.reshape(-1),
                               N_ACC=NAg, heads=heads)
    invs = _combine_inv(s_parts).reshape(-1)
    a0T = _attn_norm(dst_p, exT, invs, N_ACC=NAg, heads=heads)
    parts = _msg_pass(src_p, dst_p, a0T, featP.reshape(NP * NAg, Dp),
                      N_ACC=NAg, heads=heads, D=H)
    h1g = _combine_relu(parts, n)

    W6 = jnp.transpose(
        Wg1.reshape(NP, 2, H, NP, 2, H), (3, 0, 1, 2, 4, 5)
    ).reshape(NP, NP, Dp, Dp)
    W7 = Wg1.reshape(NP, Dp, heads, H)
    featP = _proj_heads_T(h1g, W6)
    el, er = _attn_logits_TP(h1g, W7, al1, ar1)
    elT, erT = _transpose8(el), _transpose8(er)
    exT, s_parts = _attn_stats(src_p, dst_p, elT.reshape(-1), erT.reshape(-1),
                               N_ACC=NAg, heads=heads)
    invs = _combine_inv(s_parts).reshape(-1)
    aT1 = _attn_norm(dst_p, exT, invs, a0T, N_ACC=NAg, heads=heads,
                     blend=True)
    parts = _msg_pass(src_p, dst_p, aT1, featP.reshape(NP * NAg, Dp),
                      N_ACC=NAg, heads=heads, D=H)
    h2g = _combine_relu(parts, n, h1g)

    C = Wgf.shape[1]
    featf, el, er = _proj_attn1_T(h2g, Wgf.reshape(NP, Dp, C), alf, arf)
    exT, s_parts = _attn_stats(src_p, dst_p, el.reshape(-1), er.reshape(-1),
                               N_ACC=NAg, heads=1)
    invs = _combine_inv(s_parts).reshape(-1)
    aTf = _attn_norm(dst_p, exT, invs, N_ACC=NAg, heads=1)
    partsf = _msg_pass(src_p, dst_p, aTf, featf, N_ACC=NAg, heads=1, D=C)
    out = _final(partsf[:, 0], h2g, Wres.reshape(NP, Dp, C))
    return out[:n]


# speedup vs baseline: 1.3710x; 1.1358x over previous
import functools

import jax
import jax.numpy as jnp
from jax import lax
from jax.experimental import pallas as pl
from jax.experimental.pallas import tpu as pltpu
from jax.experimental.pallas import tpu_sc as plsc

NC = 2
NS = 16
L = 16
NW = NC * NS
EB = 128
ALPHA = 0.05
NEG_SLOPE = 0.2

_SC_PARAMS = dict(
    compiler_params=pltpu.CompilerParams(
        needs_layout_passes=False, use_tc_tiling_on_sc=False
    ),
)


def _pad_up(x, m):
    return ((x + m - 1) // m) * m


def _padrows(X, N):
    return jnp.pad(X, ((0, N - X.shape[0]), (0, 0)))


def _fc(X, W, b):
    def body(x_ref, w_ref, b_ref, o_ref):
        o_ref[...] = (
            jnp.dot(x_ref[...], w_ref[...], preferred_element_type=jnp.float32)
            + b_ref[...]
        )
    n, _ = X.shape
    H = W.shape[1]
    return pl.pallas_call(
        body, out_shape=jax.ShapeDtypeStruct((n, H), jnp.float32)
    )(X, W, b.reshape(1, H))


def _proj_attn1(X, W, al, ar):
    def body(x_ref, w_ref, al_ref, ar_ref, f_ref, el_ref, er_ref):
        f = jnp.dot(x_ref[...], w_ref[...], preferred_element_type=jnp.float32)
        f_ref[...] = f
        el_ref[...] = jnp.sum(f * al_ref[...], axis=1, keepdims=True)
        er_ref[...] = jnp.sum(f * ar_ref[...], axis=1, keepdims=True)
    n = X.shape[0]
    H = W.shape[1]
    return pl.pallas_call(
        body,
        out_shape=[
            jax.ShapeDtypeStruct((n, H), jnp.float32),
            jax.ShapeDtypeStruct((n, 1), jnp.float32),
            jax.ShapeDtypeStruct((n, 1), jnp.float32),
        ],
    )(X, W, al, ar)


def _proj_attn1_T(XT, W3, al, ar):
    heads, N, H = XT.shape
    C = W3.shape[2]
    def body(x_ref, w_ref, al_ref, ar_ref, f_ref, el_ref, er_ref):
        f = jnp.dot(x_ref[0], w_ref[0], preferred_element_type=jnp.float32)
        for h in range(1, heads):
            f += jnp.dot(x_ref[h], w_ref[h], preferred_element_type=jnp.float32)
        f_ref[...] = f
        el_ref[...] = jnp.sum(f * al_ref[...], axis=1, keepdims=True)
        er_ref[...] = jnp.sum(f * ar_ref[...], axis=1, keepdims=True)
    blk = N // 8
    return pl.pallas_call(
        body,
        grid=(N // blk,),
        in_specs=[
            pl.BlockSpec((heads, blk, H), lambda i: (0, i, 0)),
            pl.BlockSpec(W3.shape, lambda i: (0, 0, 0)),
            pl.BlockSpec(al.shape, lambda i: (0, 0)),
            pl.BlockSpec(ar.shape, lambda i: (0, 0)),
        ],
        out_specs=[
            pl.BlockSpec((blk, C), lambda i: (i, 0)),
            pl.BlockSpec((blk, 1), lambda i: (i, 0)),
            pl.BlockSpec((blk, 1), lambda i: (i, 0)),
        ],
        out_shape=[
            jax.ShapeDtypeStruct((N, C), jnp.float32),
            jax.ShapeDtypeStruct((N, 1), jnp.float32),
            jax.ShapeDtypeStruct((N, 1), jnp.float32),
        ],
    )(XT, W3, al, ar)


def _proj_heads(X, W3T):
    N, K = X.shape
    heads, H = W3T.shape[0], W3T.shape[2]
    blk = N // 8
    def body(x_ref, w_ref, o_ref):
        o_ref[0] = jnp.dot(
            x_ref[...], w_ref[0], preferred_element_type=jnp.float32
        )
    return pl.pallas_call(
        body,
        grid=(heads, N // blk),
        in_specs=[
            pl.BlockSpec((blk, K), lambda h, i: (i, 0)),
            pl.BlockSpec((1, K, H), lambda h, i: (h, 0, 0)),
        ],
        out_specs=pl.BlockSpec((1, blk, H), lambda h, i: (h, i, 0)),
        out_shape=jax.ShapeDtypeStruct((heads, N, H), jnp.float32),
    )(X, W3T)


def _proj_heads_T(XT, W4):
    heads, N, H = XT.shape
    blk = N // 8
    def body(x_ref, w_ref, o_ref):
        acc = jnp.dot(x_ref[0], w_ref[0, 0], preferred_element_type=jnp.float32)
        for hi in range(1, heads):
            acc += jnp.dot(
                x_ref[hi], w_ref[0, hi], preferred_element_type=jnp.float32
            )
        o_ref[0] = acc
    return pl.pallas_call(
        body,
        grid=(heads, N // blk),
        in_specs=[
            pl.BlockSpec((heads, blk, H), lambda h, i: (0, i, 0)),
            pl.BlockSpec((1, heads, H, H), lambda h, i: (h, 0, 0, 0)),
        ],
        out_specs=pl.BlockSpec((1, blk, H), lambda h, i: (h, i, 0)),
        out_shape=jax.ShapeDtypeStruct((heads, N, H), jnp.float32),
    )(XT, W4)


def _attn_logitsP(Xp, W3T, al, ar):
    N, K = Xp.shape
    heads = W3T.shape[0]
    blk = N // 8
    def body(x_ref, w_ref, al_ref, ar_ref, el_ref, er_ref):
        VlT = jnp.sum(w_ref[...] * al_ref[...][:, None, :], axis=-1)
        VrT = jnp.sum(w_ref[...] * ar_ref[...][:, None, :], axis=-1)
        el_ref[...] = jnp.dot(x_ref[...], VlT.T,
                              preferred_element_type=jnp.float32)
        er_ref[...] = jnp.dot(x_ref[...], VrT.T,
                              preferred_element_type=jnp.float32)
    return pl.pallas_call(
        body,
        grid=(N // blk,),
        in_specs=[
            pl.BlockSpec((blk, K), lambda i: (i, 0)),
            pl.BlockSpec((heads, K, W3T.shape[2]), lambda i: (0, 0, 0)),
            pl.BlockSpec(al.shape, lambda i: (0, 0)),
            pl.BlockSpec(ar.shape, lambda i: (0, 0)),
        ],
        out_specs=[
            pl.BlockSpec((blk, heads), lambda i: (i, 0)),
            pl.BlockSpec((blk, heads), lambda i: (i, 0)),
        ],
        out_shape=[
            jax.ShapeDtypeStruct((N, heads), jnp.float32),
            jax.ShapeDtypeStruct((N, heads), jnp.float32),
        ],
    )(Xp, W3T, al, ar)


def _attn_logits_TP(XT, W7, al, ar):
    NP, N, Dp = XT.shape
    heads = W7.shape[2]
    blk = N // 8
    def body(x_ref, w_ref, al_ref, ar_ref, el_ref, er_ref):
        Vl = jnp.sum(w_ref[...] * al_ref[...][None, None, :, :], axis=-1)
        Vr = jnp.sum(w_ref[...] * ar_ref[...][None, None, :, :], axis=-1)
        accl = jnp.dot(x_ref[0], Vl[0], preferred_element_type=jnp.float32)
        accr = jnp.dot(x_ref[0], Vr[0], preferred_element_type=jnp.float32)
        for pi in range(1, NP):
            accl += jnp.dot(x_ref[pi], Vl[pi],
                            preferred_element_type=jnp.float32)
            accr += jnp.dot(x_ref[pi], Vr[pi],
                            preferred_element_type=jnp.float32)
        el_ref[...] = accl
        er_ref[...] = accr
    return pl.pallas_call(
        body,
        grid=(N // blk,),
        in_specs=[
            pl.BlockSpec((NP, blk, Dp), lambda i: (0, i, 0)),
            pl.BlockSpec(W7.shape, lambda i: (0, 0, 0, 0)),
            pl.BlockSpec(al.shape, lambda i: (0, 0)),
            pl.BlockSpec(ar.shape, lambda i: (0, 0)),
        ],
        out_specs=[
            pl.BlockSpec((blk, heads), lambda i: (i, 0)),
            pl.BlockSpec((blk, heads), lambda i: (i, 0)),
        ],
        out_shape=[
            jax.ShapeDtypeStruct((N, heads), jnp.float32),
            jax.ShapeDtypeStruct((N, heads), jnp.float32),
        ],
    )(XT, W7, al, ar)


def _transpose8(x):
    N, heads = x.shape
    def body(x_ref, o_ref):
        o_ref[...] = x_ref[...].T
    return pl.pallas_call(
        body, out_shape=jax.ShapeDtypeStruct((heads, N), jnp.float32)
    )(x)


def _combine_inv(s_parts):
    def body(s_ref, o_ref):
        o_ref[...] = 1.0 / (jnp.sum(s_ref[...], axis=0, keepdims=True) + 1e-12)
    K = s_parts.shape[1]
    return pl.pallas_call(
        body, out_shape=jax.ShapeDtypeStruct((1, K), jnp.float32)
    )(s_parts)


def _resid1(p, X):
    def body(p_ref, x_ref, o_ref):
        o_ref[...] = p_ref[0] + p_ref[1] + x_ref[...]
    n, H = X.shape
    return pl.pallas_call(
        body, out_shape=jax.ShapeDtypeStruct((n, H), jnp.float32)
    )(p, X)


def _combine_relu(p, n, res=None):
    heads, N, H = p.shape[1], p.shape[2], p.shape[3]
    blk = N // 8
    def mask(i, val):
        rows = lax.broadcasted_iota(jnp.int32, (heads, blk, H), 1) + i * blk
        return jnp.where(rows < n, val, 0.0)
    if res is None:
        def body(p_ref, o_ref):
            i = pl.program_id(0)
            o_ref[...] = mask(i, jax.nn.relu(p_ref[0] + p_ref[1]))
        ins = (p,)
        in_specs = [pl.BlockSpec((2, heads, blk, H), lambda i: (0, 0, i, 0))]
    else:
        def body(p_ref, r_ref, o_ref):
            i = pl.program_id(0)
            o_ref[...] = mask(i, jax.nn.relu(p_ref[0] + p_ref[1] + r_ref[...]))
        ins = (p, res)
        in_specs = [
            pl.BlockSpec((2, heads, blk, H), lambda i: (0, 0, i, 0)),
            pl.BlockSpec((heads, blk, H), lambda i: (0, i, 0)),
        ]
    return pl.pallas_call(
        body,
        grid=(N // blk,),
        in_specs=in_specs,
        out_specs=pl.BlockSpec((heads, blk, H), lambda i: (0, i, 0)),
        out_shape=jax.ShapeDtypeStruct((heads, N, H), jnp.float32),
    )(*ins)


def _final(pf, XT, Wres3):
    heads, N, H = XT.shape
    C = Wres3.shape[2]
    def body(p_ref, x_ref, w_ref, o_ref):
        y = p_ref[0] + p_ref[1]
        for h in range(heads):
            y += jnp.dot(x_ref[h], w_ref[h], preferred_element_type=jnp.float32)
        nrm = jnp.sqrt(jnp.sum(y * y, axis=1, keepdims=True))
        o_ref[...] = y / jnp.maximum(nrm, 1e-12)
    blk = N // 8
    return pl.pallas_call(
        body,
        grid=(N // blk,),
        in_specs=[
            pl.BlockSpec((2, blk, C), lambda i: (0, i, 0)),
            pl.BlockSpec((heads, blk, H), lambda i: (0, i, 0)),
            pl.BlockSpec(Wres3.shape, lambda i: (0, 0, 0)),
        ],
        out_specs=pl.BlockSpec((blk, C), lambda i: (i, 0)),
        out_shape=jax.ShapeDtypeStruct((N, C), jnp.float32),
    )(pf, XT, Wres3)


def _mesh():
    return plsc.VectorSubcoreMesh(
        core_axis_name="c", subcore_axis_name="s", num_cores=NC, num_subcores=NS
    )


@functools.partial(jax.jit, static_argnames=("N_ACC", "heads"))
def _attn_stats(src, dst, elT, erT, *, N_ACC, heads):
    E_pad = src.shape[0]
    M = E_pad // NW
    nblk = M // EB

    @functools.partial(
        pl.kernel,
        out_type=[
            jax.ShapeDtypeStruct((heads * E_pad,), jnp.float32),
            jax.ShapeDtypeStruct((NW, heads * N_ACC), jnp.float32),
        ],
        mesh=_mesh(),
        scratch_types=[
            pltpu.VMEM((N_ACC,), jnp.float32),
            pltpu.VMEM((N_ACC,), jnp.float32),
            pltpu.VMEM((heads * N_ACC,), jnp.float32),
            pltpu.VMEM((2, EB), jnp.int32),
            pltpu.VMEM((2, EB), jnp.int32),
            pltpu.VMEM((2, EB), jnp.float32),
            pltpu.SemaphoreType.DMA,
            pltpu.SemaphoreType.DMA,
            pltpu.SemaphoreType.DMA,
            pltpu.SemaphoreType.DMA,
        ],
        **_SC_PARAMS,
    )
    def k(src_h, dst_h, el_h, er_h, exT_h, sp_h, elv, erv, sv, srcb, dstb,
          exb, ls0, ls1, ws0, ws1):
        lsem = (ls0, ls1)
        wsem = (ws0, ws1)
        c = lax.axis_index("c")
        s = lax.axis_index("s")
        wid = s * NC + c
        base0 = wid * M

        @pl.loop(0, heads * N_ACC // L)
        def _zero(i):
            sv[pl.ds(i * L, L)] = jnp.zeros((L,), jnp.float32)

        for h in range(heads):
            pltpu.sync_copy(el_h.at[pl.ds(h * N_ACC, N_ACC)], elv)
            pltpu.sync_copy(er_h.at[pl.ds(h * N_ACC, N_ACC)], erv)

            def loads(t, k_):
                base = base0 + k_ * EB
                pltpu.async_copy(src_h.at[pl.ds(base, EB)],
                                 srcb.at[t], lsem[t])
                pltpu.async_copy(dst_h.at[pl.ds(base, EB)],
                                 dstb.at[t], lsem[t])

            def wait_loads(t):
                pltpu.make_async_copy(
                    src_h.at[pl.ds(0, EB)], srcb.at[t], lsem[t]).wait()
                pltpu.make_async_copy(
                    dst_h.at[pl.ds(0, EB)], dstb.at[t], lsem[t]).wait()

            loads(0, 0)

            @pl.loop(0, nblk, step=2)
            def _blk(b):
                for t in range(2):
                    k_ = b + t
                    wait_loads(t)
                    @pl.when(k_ >= 2)
                    def _drain():
                        pltpu.make_async_copy(
                            exb.at[t], exT_h.at[pl.ds(0, EB)], wsem[t]).wait()

                    @pl.when(k_ + 1 < nblk)
                    def _pref():
                        loads(1 - t, k_ + 1)

                    for j in range(EB // L):
                        sc_ = srcb[t, pl.ds(j * L, L)]
                        dc_ = dstb[t, pl.ds(j * L, L)]
                        e = (plsc.load_gather(elv, [sc_])
                             + plsc.load_gather(erv, [dc_]))
                        e = jnp.maximum(e, NEG_SLOPE * e)
                        ex = jnp.exp(e)
                        exb[t, pl.ds(j * L, L)] = ex
                        plsc.addupdate_scatter(sv, [dc_ + (h * N_ACC)], ex)
                    g_ = wid * nblk + k_
                    pltpu.async_copy(
                        exb.at[t], exT_h.at[pl.ds((g_ * heads + h) * EB, EB)],
                        wsem[t])

            for t in range(2):
                pltpu.make_async_copy(
                    exb.at[t], exT_h.at[pl.ds(0, EB)], wsem[t]).wait()

        pltpu.sync_copy(sv, sp_h.at[wid])

    return k(src, dst, elT, erT)


@functools.partial(jax.jit, static_argnames=("N_ACC", "heads", "blend"))
def _attn_norm(dst, exT, invs, a0T=None, *, N_ACC, heads, blend=False):
    E_pad = dst.shape[0]
    M = E_pad // NW
    nblk = M // EB

    HB = heads * EB

    @functools.partial(
        pl.kernel,
        out_type=jax.ShapeDtypeStruct((heads * E_pad,), jnp.float32),
        mesh=_mesh(),
        scratch_types=[
            pltpu.VMEM((heads, N_ACC), jnp.float32),
            pltpu.VMEM((2, EB), jnp.int32),
            pltpu.VMEM((2, HB), jnp.float32),
            pltpu.VMEM((2, HB), jnp.float32),
            pltpu.VMEM((2, HB), jnp.float32),
            pltpu.SemaphoreType.DMA,
            pltpu.SemaphoreType.DMA,
            pltpu.SemaphoreType.DMA,
            pltpu.SemaphoreType.DMA,
        ],
        **_SC_PARAMS,
    )
    def k(*refs):
        if blend:
            (dst_h, exT_h, invs_h, a0T_h) = refs[:4]
            (aT_h, invs_v, dstb, exb, a0b, ab, ls0, ls1, ws0, ws1) = refs[4:]
        else:
            (dst_h, exT_h, invs_h) = refs[:3]
            (aT_h, invs_v, dstb, exb, a0b, ab, ls0, ls1, ws0, ws1) = refs[3:]
            a0T_h = None
        lsem = (ls0, ls1)
        wsem = (ws0, ws1)
        c = lax.axis_index("c")
        s = lax.axis_index("s")
        wid = s * NC + c
        g0_ = wid * nblk

        for h in range(heads):
            pltpu.sync_copy(invs_h.at[pl.ds(h * N_ACC, N_ACC)],
                            invs_v.at[h])

        def loads(t, k_):
            pltpu.async_copy(dst_h.at[pl.ds((g0_ + k_) * EB, EB)],
                             dstb.at[t], lsem[t])
            pltpu.async_copy(exT_h.at[pl.ds((g0_ + k_) * HB, HB)],
                             exb.at[t], lsem[t])
            if blend:
                pltpu.async_copy(a0T_h.at[pl.ds((g0_ + k_) * HB, HB)],
                                 a0b.at[t], lsem[t])

        def wait_loads(t):
            pltpu.make_async_copy(
                dst_h.at[pl.ds(0, EB)], dstb.at[t], lsem[t]).wait()
            pltpu.make_async_copy(
                exT_h.at[pl.ds(0, HB)], exb.at[t], lsem[t]).wait()
            if blend:
                pltpu.make_async_copy(
                    a0T_h.at[pl.ds(0, HB)], a0b.at[t], lsem[t]).wait()

        loads(0, 0)

        @pl.loop(0, nblk, step=2)
        def _blk(b):
            for t in range(2):
                k_ = b + t
                wait_loads(t)
                @pl.when(k_ >= 2)
                def _drain():
                    pltpu.make_async_copy(
                        ab.at[t], aT_h.at[pl.ds(0, HB)], wsem[t]).wait()

                @pl.when(k_ + 1 < nblk)
                def _pref():
                    loads(1 - t, k_ + 1)

                for j in range(EB // L):
                    dc_ = dstb[t, pl.ds(j * L, L)]
                    for h in range(heads):
                        iv = plsc.load_gather(
                            invs_v, [jnp.full((L,), h, jnp.int32), dc_])
                        a = exb[t, pl.ds(h * EB + j * L, L)] * iv
                        if blend:
                            a = (a * (1.0 - ALPHA)
                                 + ALPHA * a0b[t, pl.ds(h * EB + j * L, L)])
                        ab[t, pl.ds(h * EB + j * L, L)] = a
                pltpu.async_copy(ab.at[t],
                                 aT_h.at[pl.ds((g0_ + k_) * HB, HB)],
                                 wsem[t])

        for t in range(2):
            pltpu.make_async_copy(
                ab.at[t], aT_h.at[pl.ds(0, HB)], wsem[t]).wait()

    args = (dst, exT, invs) + ((a0T,) if blend else ())
    return k(*args)


@functools.partial(jax.jit, static_argnames=("N_ACC", "heads", "D"))
def _msg_pass(src, dst, aT, featflat, *, N_ACC, heads, D):
    E_pad = src.shape[0]
    M = E_pad // NW
    nblk = M // EB
    rpt = N_ACC // NS
    HP = 2 if heads == 8 else 1
    NP = heads // HP
    Dp = HP * D

    scratch = [
        pltpu.VMEM_SHARED((N_ACC, Dp), jnp.float32),
        pltpu.VMEM((2, EB), jnp.int32),
        pltpu.VMEM((2, EB), jnp.int32),
        pltpu.VMEM((2, EB), jnp.int32),
        pltpu.VMEM((2, EB), jnp.int32),
        pltpu.VMEM((2, HP * EB), jnp.float32),
        pltpu.VMEM((2, EB, Dp), jnp.float32),
        pltpu.VMEM((32, Dp), jnp.float32),
        pltpu.SemaphoreType.DMA,
        pltpu.SemaphoreType.DMA,
        pltpu.SemaphoreType.DMA,
        pltpu.SemaphoreType.DMA,
        pltpu.SemaphoreType.DMA,
        pltpu.SemaphoreType.DMA,
    ]

    @functools.partial(
        pl.kernel,
        out_type=jax.ShapeDtypeStruct((NC, NP, N_ACC, Dp), jnp.float32),
        mesh=_mesh(), scratch_types=scratch,
        **_SC_PARAMS,
    )
    def k(src_h, dst_h, aT_h, feat_h, out_h, acc, srcb, dstb, idxb, sdst,
          ab, rows, zv, ls0, ls1, gs0, gs1, ss0, ss1):
        lsem = (ls0, ls1)
        gsem = (gs0, gs1)
        ssem = (ss0, ss1)

        c = lax.axis_index("c")
        s = lax.axis_index("s")
        wid = s * NC + c
        base0 = wid * M

        @pl.loop(0, 32)
        def _zfill(i):
            for j in range(Dp // L):
                zv[i, pl.ds(j * L, L)] = jnp.zeros((L,), jnp.float32)

        for p in range(NP):
            off = 0
            while off < rpt:
                csz = min(32, rpt - off)
                pltpu.sync_copy(
                    zv.at[pl.ds(0, csz), :],
                    acc.at[pl.ds(s * rpt + off, csz), :],
                )
                off += csz
            plsc.subcore_barrier()

            def loads(t, k_):
                base = base0 + k_ * EB
                pltpu.async_copy(src_h.at[pl.ds(base, EB)],
                                 srcb.at[t], lsem[t])
                pltpu.async_copy(dst_h.at[pl.ds(base, EB)],
                                 dstb.at[t], lsem[t])
                eo = ((wid * nblk + k_) * heads + p * HP) * EB
                pltpu.async_copy(aT_h.at[pl.ds(eo, HP * EB)],
                                 ab.at[t], lsem[t])

            def wait_loads(t):
                pltpu.make_async_copy(
                    src_h.at[pl.ds(0, EB)], srcb.at[t], lsem[t]).wait()
                pltpu.make_async_copy(
                    dst_h.at[pl.ds(0, EB)], dstb.at[t], lsem[t]).wait()
                pltpu.make_async_copy(
                    aT_h.at[pl.ds(0, HP * EB)], ab.at[t], lsem[t]).wait()

            loads(0, 0)

            @pl.loop(0, nblk, step=2)
            def _blk(b):
                for t in range(2):
                    k_ = b + t
                    wait_loads(t)
                    @pl.when(k_ >= 2)
                    def _drain():
                        pltpu.make_async_copy(
                            rows.at[t], acc.at[sdst.at[t]], ssem[t]).wait()
                    if NP > 1:
                        for j in range(EB // L):
                            idxb[t, pl.ds(j * L, L)] = (
                                srcb[t, pl.ds(j * L, L)] + (p * N_ACC)
                            )
                        gref = idxb.at[t]
                    else:
                        gref = srcb.at[t]
                    gd = pltpu.async_copy(feat_h.at[gref], rows.at[t],
                                          gsem[t])

                    @pl.when(k_ + 1 < nblk)
                    def _pref():
                        loads(1 - t, k_ + 1)

                    for j in range(EB // L):
                        sdst[t, pl.ds(j * L, L)] = dstb[t, pl.ds(j * L, L)]
                    gd.wait()

                    rv = rows.at[t]

                    @pl.loop(0, EB, unroll=8)
                    def _scale(bb):
                        for hh in range(HP):
                            av = plsc.load_gather(
                                ab.at[t],
                                [jnp.full((L,), bb, jnp.int32) + (hh * EB)])
                            for j in range(D // L):
                                o = hh * D + j * L
                                rv[bb, pl.ds(o, L)] = rv[bb, pl.ds(o, L)] * av

                    pltpu.async_copy(rows.at[t], acc.at[sdst.at[t]],
                                     ssem[t], add=True)

            for t in range(2):
                pltpu.make_async_copy(
                    rows.at[t], acc.at[sdst.at[t]], ssem[t]).wait()

            plsc.subcore_barrier()
            pltpu.sync_copy(
                acc.at[pl.ds(s * rpt, rpt), :],
                out_h.at[c, p, pl.ds(s * rpt, rpt), :],
            )
            plsc.subcore_barrier()

    return k(src, dst, aT, featflat)


def _pad_edges(src, dst, n):
    E = src.shape[0]
    E_pad = _pad_up(E, NW * EB * 2)
    pad = E_pad - E
    src_p = jnp.concatenate([src, jnp.zeros((pad,), jnp.int32)])
    dst_p = jnp.concatenate([dst, jnp.full((pad,), n, jnp.int32)])
    return src_p, dst_p


def _gat_layer1(featp, elf, erf, src, dst, n, N_ACC):
    src_p, dst_p = _pad_edges(src, dst, n)
    exT, s_parts = _attn_stats(src_p, dst_p, elf, erf, N_ACC=N_ACC, heads=1)
    invs = _combine_inv(s_parts).reshape(-1)
    aT = _attn_norm(dst_p, exT, invs, N_ACC=N_ACC, heads=1)
    return _msg_pass(src_p, dst_p, aT, featp,
                     N_ACC=N_ACC, heads=1, D=featp.shape[1])


def kernel(features_0, features_1, features_2, W_fc0, b_fc0, W_fc1, b_fc1,
           W_fc2, b_fc2, Wp0, alp0, arp0, Wp1, alp1, arp1, Wg0, al0, ar0,
           Wg1, al1, ar1, Wgf, alf, arf, Wres, hg0_src, hg0_dst, hg1_src,
           hg1_dst, g_src, g_dst):
    H = 64
    heads = 8
    n = 10000
    NA0 = _pad_up(5000 + 1, 128)
    NA1 = _pad_up(8000 + 1, 128)
    NAg = _pad_up(n + 1, 128)

    h0 = _fc(features_0, W_fc0, b_fc0)
    h1_ = _fc(features_1, W_fc1, b_fc1)
    h2_ = _fc(features_2, W_fc2, b_fc2)

    feat, el, er = _proj_attn1(_padrows(h0, NA0), Wp0, alp0, arp0)
    p = _gat_layer1(feat, el.reshape(-1), er.reshape(-1),
                    hg0_src, hg0_dst, 5000, NA0)
    h0n = _resid1(p[:, 0, :5000], h0)

    hcat = jnp.concatenate([h0n, h1_], axis=0)
    feat, el, er = _proj_attn1(_padrows(hcat, NA1), Wp1, alp1, arp1)
    p = _gat_layer1(feat, el.reshape(-1), er.reshape(-1),
                    hg1_src, hg1_dst, 8000, NA1)
    h1n = _resid1(p[:, 0, 5000:8000], h1_)

    hfp = _padrows(jnp.concatenate([h0n, h1n, h2_], axis=0), NAg)
    src_p, dst_p = _pad_edges(g_src, g_dst, n)

    NP = heads // 2
    Dp = 2 * H
    W3T = jnp.transpose(Wg0.reshape(H, heads, H), (1, 0, 2))
    W5 = jnp.transpose(Wg0.reshape(H, NP, 2, H), (1, 0, 2, 3)).reshape(
        NP, H, Dp)
    featP = _proj_heads(hfp, W5)
    el, er = _attn_logitsP(hfp, W3T, al0, ar0)
    elT, erT = _transpose8(el), _transpose8(er)
    exT, s_parts = _attn_stats(src_p, dst_p, elT.reshape(-1), erT.reshape(-1),
                               N_ACC=NAg, heads=heads)
    invs = _combine_inv(s_parts).reshape(-1)
    a0T = _attn_norm(dst_p, exT, invs, N_ACC=NAg, heads=heads)
    parts = _msg_pass(src_p, dst_p, a0T, featP.reshape(NP * NAg, Dp),
                      N_ACC=NAg, heads=heads, D=H)
    h1g = _combine_relu(parts, n)

    W6 = jnp.transpose(
        Wg1.reshape(NP, 2, H, NP, 2, H), (3, 0, 1, 2, 4, 5)
    ).reshape(NP, NP, Dp, Dp)
    W7 = Wg1.reshape(NP, Dp, heads, H)
    featP = _proj_heads_T(h1g, W6)
    el, er = _attn_logits_TP(h1g, W7, al1, ar1)
    elT, erT = _transpose8(el), _transpose8(er)
    exT, s_parts = _attn_stats(src_p, dst_p, elT.reshape(-1), erT.reshape(-1),
                               N_ACC=NAg, heads=heads)
    invs = _combine_inv(s_parts).reshape(-1)
    aT1 = _attn_norm(dst_p, exT, invs, a0T, N_ACC=NAg, heads=heads,
                     blend=True)
    parts = _msg_pass(src_p, dst_p, aT1, featP.reshape(NP * NAg, Dp),
                      N_ACC=NAg, heads=heads, D=H)
    h2g = _combine_relu(parts, n, h1g)

    C = Wgf.shape[1]
    featf, el, er = _proj_attn1_T(h2g, Wgf.reshape(NP, Dp, C), alf, arf)
    exT, s_parts = _attn_stats(src_p, dst_p, el.reshape(-1), er.reshape(-1),
                               N_ACC=NAg, heads=1)
    invs = _combine_inv(s_parts).reshape(-1)
    aTf = _attn_norm(dst_p, exT, invs, N_ACC=NAg, heads=1)
    partsf = _msg_pass(src_p, dst_p, aTf, featf, N_ACC=NAg, heads=1, D=C)
    out = _final(partsf[:, 0], h2g, Wres.reshape(NP, Dp, C))
    return out[:n]
